# Initial kernel scaffold; baseline (speedup 1.0000x reference)
import jax
import jax.numpy as jnp
from jax.experimental import pallas as pl


def kernel(input_ids, weight):
    raise NotImplementedError("write your pallas kernel here")


# speedup vs baseline: 2.9768x; 2.9768x over previous
import functools

import jax
import jax.numpy as jnp
from jax import lax
from jax.experimental import pallas as pl
from jax.experimental.pallas import tpu as pltpu
from jax.experimental.pallas import tpu_sc as plsc

EMBED = 128
CHUNK = 128


def _gather_body(nc, nchunks, ids_hbm, table_hbm, out_hbm, idx_v, rows_v, sem):
    per_w = nchunks * CHUNK
    wid = lax.axis_index("s") * nc + lax.axis_index("c")
    base = wid * per_w
    pltpu.sync_copy(ids_hbm.at[wid], idx_v)

    @pl.loop(0, nchunks)
    def _(j):
        pltpu.async_copy(table_hbm.at[idx_v.at[j]], rows_v, sem).wait()
        pltpu.sync_copy(rows_v, out_hbm.at[pl.ds(base + j * CHUNK, CHUNK)])


def kernel(input_ids, weight):
    batch, hist = input_ids.shape
    n = batch * hist
    info = plsc.get_sparse_core_info()
    nw = info.num_cores * info.num_subcores
    assert n % (nw * CHUNK) == 0
    nchunks = n // (nw * CHUNK)

    ids = input_ids.reshape(nw, nchunks, CHUNK).astype(jnp.int32)
    mesh = plsc.VectorSubcoreMesh(core_axis_name="c", subcore_axis_name="s")

    run = pl.kernel(
        functools.partial(_gather_body, info.num_cores, nchunks),
        out_type=jax.ShapeDtypeStruct((n, EMBED), jnp.float32),
        mesh=mesh,
        scratch_types=[
            pltpu.VMEM((nchunks, CHUNK), jnp.int32),
            pltpu.VMEM((CHUNK, EMBED), jnp.float32),
            pltpu.SemaphoreType.DMA,
        ],
    )
    out = run(ids, weight)
    return out.reshape(batch, hist, EMBED)


# speedup vs baseline: 3.3387x; 1.1216x over previous
import functools

import jax
import jax.numpy as jnp
from jax import lax
from jax.experimental import pallas as pl
from jax.experimental.pallas import tpu as pltpu
from jax.experimental.pallas import tpu_sc as plsc

EMBED = 128
CHUNK = 128
KG = 2
ROWS = KG * CHUNK


def _gather_body(nc, ngroups, ids_hbm, table_hbm, out_hbm,
                 idx_v, rows0, rows1, gs0, gs1, ws0, ws1):
    wid = lax.axis_index("s") * nc + lax.axis_index("c")
    base = wid * ngroups * ROWS
    pltpu.sync_copy(ids_hbm.at[wid], idx_v)

    def fire_gather(g, buf, sem):
        for k in range(KG):
            pltpu.async_copy(table_hbm.at[idx_v.at[KG * g + k]],
                             buf.at[pl.ds(k * CHUNK, CHUNK)], sem)

    def drain_gather(buf, sem):
        pltpu.make_async_copy(out_hbm.at[pl.ds(0, ROWS)], buf, sem).wait()

    def fire_write(g, buf, sem):
        pltpu.async_copy(buf, out_hbm.at[pl.ds(base + g * ROWS, ROWS)], sem)

    def drain_write(sem):
        pltpu.make_async_copy(rows0, out_hbm.at[pl.ds(base, ROWS)], sem).wait()

    fire_gather(0, rows0, gs0)

    @pl.loop(0, ngroups, step=2)
    def _(g):
        @pl.when(g + 1 < ngroups)
        def _():
            @pl.when(g >= 2)
            def _():
                drain_write(ws1)
            fire_gather(g + 1, rows1, gs1)

        drain_gather(rows0, gs0)
        fire_write(g, rows0, ws0)

        @pl.when(g + 1 < ngroups)
        def _():
            @pl.when(g + 2 < ngroups)
            def _():
                drain_write(ws0)
                fire_gather(g + 2, rows0, gs0)
            drain_gather(rows1, gs1)
            fire_write(g + 1, rows1, ws1)

    drain_write(ws0)
    drain_write(ws1)


def kernel(input_ids, weight):
    batch, hist = input_ids.shape
    n = batch * hist
    info = plsc.get_sparse_core_info()
    nw = info.num_cores * info.num_subcores
    assert n % (nw * ROWS) == 0
    ngroups = n // (nw * ROWS)
    nchunks = ngroups * KG

    ids = input_ids.reshape(nw, nchunks, CHUNK).astype(jnp.int32)
    mesh = plsc.VectorSubcoreMesh(core_axis_name="c", subcore_axis_name="s")

    run = pl.kernel(
        functools.partial(_gather_body, info.num_cores, ngroups),
        out_type=jax.ShapeDtypeStruct((n, EMBED), jnp.float32),
        mesh=mesh,
        scratch_types=[
            pltpu.VMEM((nchunks, CHUNK), jnp.int32),
            pltpu.VMEM((ROWS, EMBED), jnp.float32),
            pltpu.VMEM((ROWS, EMBED), jnp.float32),
            pltpu.SemaphoreType.DMA,
            pltpu.SemaphoreType.DMA,
            pltpu.SemaphoreType.DMA,
            pltpu.SemaphoreType.DMA,
        ],
    )
    out = run(ids, weight)
    return out.reshape(batch, hist, EMBED)


# speedup vs baseline: 3.3529x; 1.0043x over previous
import functools

import jax
import jax.numpy as jnp
from jax import lax
from jax.experimental import pallas as pl
from jax.experimental.pallas import tpu as pltpu
from jax.experimental.pallas import tpu_sc as plsc

EMBED = 128
CHUNK = 128
NB = 5
AHEAD = 3


def _gather_body(nc, nchunks, ids_hbm, table_hbm, out_hbm, idx_v, *scratch):
    rows = scratch[:NB]
    gs = scratch[NB:2 * NB]
    ws = scratch[2 * NB:]
    wid = lax.axis_index("s") * nc + lax.axis_index("c")
    base = wid * nchunks * CHUNK
    pltpu.sync_copy(ids_hbm.at[wid], idx_v)

    def fire_gather(t, b):
        pltpu.async_copy(table_hbm.at[idx_v.at[t]], rows[b], gs[b])

    def drain_gather(b):
        pltpu.make_async_copy(out_hbm.at[pl.ds(0, CHUNK)], rows[b], gs[b]).wait()

    def fire_write(t, b):
        pltpu.async_copy(rows[b], out_hbm.at[pl.ds(base + t * CHUNK, CHUNK)],
                         ws[b])

    def drain_write(b):
        pltpu.make_async_copy(rows[b], out_hbm.at[pl.ds(base, CHUNK)],
                              ws[b]).wait()

    for t in range(AHEAD):
        fire_gather(t, t)

    @pl.loop(0, nchunks, step=NB)
    def _(g):
        for i in range(NB):
            t = g + i
            bf = (i + AHEAD) % NB

            @pl.when(t + AHEAD < nchunks)
            def _():
                @pl.when(t >= NB - AHEAD)
                def _():
                    drain_write(bf)
                fire_gather(t + AHEAD, bf)

            drain_gather(i)
            fire_write(t, i)

    for b in range(NB):
        drain_write(b)


def kernel(input_ids, weight):
    batch, hist = input_ids.shape
    n = batch * hist
    info = plsc.get_sparse_core_info()
    nw = info.num_cores * info.num_subcores
    assert n % (nw * CHUNK * NB) == 0
    nchunks = n // (nw * CHUNK)

    ids = input_ids.reshape(nw, nchunks, CHUNK).astype(jnp.int32)
    mesh = plsc.VectorSubcoreMesh(core_axis_name="c", subcore_axis_name="s")

    run = pl.kernel(
        functools.partial(_gather_body, info.num_cores, nchunks),
        out_type=jax.ShapeDtypeStruct((n, EMBED), jnp.float32),
        mesh=mesh,
        scratch_types=(
            [pltpu.VMEM((nchunks, CHUNK), jnp.int32)]
            + [pltpu.VMEM((CHUNK, EMBED), jnp.float32) for _ in range(NB)]
            + [pltpu.SemaphoreType.DMA for _ in range(2 * NB)]
        ),
    )
    out = run(ids, weight)
    return out.reshape(batch, hist, EMBED)


# speedup vs baseline: 5.9688x; 1.7802x over previous
import functools

import jax
import jax.numpy as jnp
from jax import lax
from jax.experimental import pallas as pl
from jax.experimental.pallas import tpu as pltpu
from jax.experimental.pallas import tpu_sc as plsc

EMBED = 128
NB = 8
AHEAD = 6


def _gather_body(nc, per_w, hist, ids_hbm, table_hbm, out_hbm, idx_v, *scratch):
    rows = scratch[:NB]
    gs = scratch[NB:2 * NB]
    ws = scratch[2 * NB:]
    wid = lax.axis_index("s") * nc + lax.axis_index("c")
    base = wid * per_w
    pltpu.sync_copy(ids_hbm.at[wid], idx_v)

    def fire_gather(t, b):
        pltpu.async_copy(table_hbm.at[idx_v.at[t]], rows[b], gs[b])

    def drain_gather(b):
        pltpu.make_async_copy(out_hbm.at[0], rows[b], gs[b]).wait()

    def fire_write(t, b):
        pltpu.async_copy(rows[b], out_hbm.at[base + t], ws[b])

    def drain_write(b):
        pltpu.make_async_copy(rows[b], out_hbm.at[base], ws[b]).wait()

    for t in range(AHEAD):
        fire_gather(t, t)

    @pl.loop(0, per_w, step=NB)
    def _(g):
        for i in range(NB):
            t = g + i
            bf = (i + AHEAD) % NB

            @pl.when(t + AHEAD < per_w)
            def _():
                @pl.when(t >= NB - AHEAD)
                def _():
                    drain_write(bf)
                fire_gather(t + AHEAD, bf)

            drain_gather(i)
            fire_write(t, i)

    for b in range(NB):
        drain_write(b)


def kernel(input_ids, weight):
    batch, hist = input_ids.shape
    info = plsc.get_sparse_core_info()
    nw = info.num_cores * info.num_subcores
    assert batch % (nw * NB) == 0
    per_w = batch // nw

    ids = input_ids.reshape(nw, per_w, hist).astype(jnp.int32)
    mesh = plsc.VectorSubcoreMesh(core_axis_name="c", subcore_axis_name="s")

    run = pl.kernel(
        functools.partial(_gather_body, info.num_cores, per_w, hist),
        out_type=jax.ShapeDtypeStruct((batch, hist, EMBED), jnp.float32),
        mesh=mesh,
        scratch_types=(
            [pltpu.VMEM((per_w, hist), jnp.int32)]
            + [pltpu.VMEM((hist, EMBED), jnp.float32) for _ in range(NB)]
            + [pltpu.SemaphoreType.DMA for _ in range(2 * NB)]
        ),
    )
    return run(ids, weight)


# speedup vs baseline: 5.9753x; 1.0011x over previous
import functools

import jax
import jax.numpy as jnp
from jax import lax
from jax.experimental import pallas as pl
from jax.experimental.pallas import tpu as pltpu
from jax.experimental.pallas import tpu_sc as plsc

EMBED = 128
NB = 8
AHEAD = 6


def _gather_body(nc, per_w, hist, ids_hbm, table_hbm, out_hbm, idx_v, *scratch):
    rows = scratch[:NB]
    gs = scratch[NB:2 * NB]
    ws = scratch[2 * NB:]
    wid = lax.axis_index("s") * nc + lax.axis_index("c")
    base = wid * per_w
    pltpu.sync_copy(ids_hbm.at[wid], idx_v)

    def fire_gather(t, b):
        pltpu.async_copy(table_hbm.at[idx_v.at[t]], rows[b], gs[b])

    def drain_gather(b):
        pltpu.make_async_copy(out_hbm.at[0], rows[b], gs[b]).wait()

    def fire_write(t, b):
        pltpu.async_copy(rows[b], out_hbm.at[base + t], ws[b])

    def drain_write(b):
        pltpu.make_async_copy(rows[b], out_hbm.at[base], ws[b]).wait()

    for t in range(AHEAD):
        fire_gather(t, t)

    @pl.loop(0, per_w, step=NB)
    def _(g):
        for i in range(NB):
            t = g + i
            bf = (i + AHEAD) % NB

            @pl.when(t + AHEAD < per_w)
            def _():
                @pl.when(t >= NB - AHEAD)
                def _():
                    drain_write(bf)
                fire_gather(t + AHEAD, bf)

            drain_gather(i)
            fire_write(t, i)

    for b in range(NB):
        drain_write(b)


def kernel(input_ids, weight):
    batch, hist = input_ids.shape
    info = plsc.get_sparse_core_info()
    nw = info.num_cores * info.num_subcores
    assert batch % (nw * NB) == 0
    per_w = batch // nw

    ids = input_ids.reshape(nw, per_w, hist).astype(jnp.int32)
    mesh = plsc.VectorSubcoreMesh(core_axis_name="c", subcore_axis_name="s")

    run = pl.kernel(
        functools.partial(_gather_body, info.num_cores, per_w, hist),
        out_type=jax.ShapeDtypeStruct((batch, hist, EMBED), jnp.float32),
        mesh=mesh,
        compiler_params=pltpu.CompilerParams(use_tc_tiling_on_sc=True),
        scratch_types=(
            [pltpu.VMEM((per_w, hist), jnp.int32)]
            + [pltpu.VMEM((hist, EMBED), jnp.float32) for _ in range(NB)]
            + [pltpu.SemaphoreType.DMA for _ in range(2 * NB)]
        ),
    )
    return run(ids, weight)


# speedup vs baseline: 10.6428x; 1.7811x over previous
import functools

import jax
import jax.numpy as jnp
from jax import lax
from jax.experimental import pallas as pl
from jax.experimental.pallas import tpu as pltpu
from jax.experimental.pallas import tpu_sc as plsc

EMBED = 128
BLOCK = 128
NB = 5
AHEAD = 3


def _gather_body(nc, hist, ids_hbm, table_hbm, out_hbm, idx_v, *scratch):
    rows = scratch[:NB]
    gs = scratch[NB:2 * NB]
    ws = scratch[2 * NB:]
    wid = lax.axis_index("s") * nc + lax.axis_index("c")
    base = wid * BLOCK
    pltpu.sync_copy(ids_hbm.at[wid], idx_v)

    def fire_gather(t, b):
        pltpu.async_copy(table_hbm.at[idx_v.at[t]], rows[b], gs[b])

    def drain_gather(b):
        pltpu.make_async_copy(out_hbm.at[0, pl.ds(base, BLOCK)], rows[b],
                              gs[b]).wait()

    def fire_write(t, b):
        pltpu.async_copy(rows[b], out_hbm.at[t, pl.ds(base, BLOCK)], ws[b])

    def drain_write(b):
        pltpu.make_async_copy(rows[b], out_hbm.at[0, pl.ds(base, BLOCK)],
                              ws[b]).wait()

    for t in range(AHEAD):
        fire_gather(t, t)

    @pl.loop(0, hist, step=NB)
    def _(g):
        for i in range(NB):
            t = g + i
            bf = (i + AHEAD) % NB

            @pl.when(t + AHEAD < hist)
            def _():
                @pl.when(t >= NB - AHEAD)
                def _():
                    drain_write(bf)
                fire_gather(t + AHEAD, bf)

            drain_gather(i)
            fire_write(t, i)

    for b in range(NB):
        drain_write(b)


def kernel(input_ids, weight):
    batch, hist = input_ids.shape
    info = plsc.get_sparse_core_info()
    nw = info.num_cores * info.num_subcores
    assert batch % (nw * BLOCK) == 0 or batch == nw * BLOCK
    assert hist % NB == 0

    ids_w = jnp.transpose(
        input_ids.astype(jnp.int32).reshape(nw, BLOCK, hist), (0, 2, 1))
    mesh = plsc.VectorSubcoreMesh(core_axis_name="c", subcore_axis_name="s")

    run = pl.kernel(
        functools.partial(_gather_body, info.num_cores, hist),
        out_type=jax.ShapeDtypeStruct((hist, batch, EMBED), jnp.float32),
        mesh=mesh,
        scratch_types=(
            [pltpu.VMEM((hist, BLOCK), jnp.int32)]
            + [pltpu.VMEM((BLOCK, EMBED), jnp.float32) for _ in range(NB)]
            + [pltpu.SemaphoreType.DMA for _ in range(2 * NB)]
        ),
    )
    out = run(ids_w, weight)
    return jnp.transpose(out, (1, 0, 2))


# speedup vs baseline: 10.6767x; 1.0032x over previous
import functools

import jax
import jax.numpy as jnp
from jax import lax
from jax.experimental import pallas as pl
from jax.experimental.pallas import tpu as pltpu
from jax.experimental.pallas import tpu_sc as plsc

EMBED = 128
BLOCK = 128
NB = 5
AHEAD = 4


def _gather_body(nc, hist, ids_hbm, table_hbm, out_hbm, idx_v, *scratch):
    rows = scratch[:NB]
    gs = scratch[NB:2 * NB]
    ws = scratch[2 * NB:]
    wid = lax.axis_index("s") * nc + lax.axis_index("c")
    base = wid * BLOCK
    pltpu.sync_copy(ids_hbm.at[wid], idx_v)

    def fire_gather(t, b):
        pltpu.async_copy(table_hbm.at[idx_v.at[t]], rows[b], gs[b])

    def drain_gather(b):
        pltpu.make_async_copy(out_hbm.at[0, pl.ds(base, BLOCK)], rows[b],
                              gs[b]).wait()

    def fire_write(t, b):
        pltpu.async_copy(rows[b], out_hbm.at[t, pl.ds(base, BLOCK)], ws[b])

    def drain_write(b):
        pltpu.make_async_copy(rows[b], out_hbm.at[0, pl.ds(base, BLOCK)],
                              ws[b]).wait()

    for t in range(AHEAD):
        fire_gather(t, t)

    @pl.loop(0, hist, step=NB)
    def _(g):
        for i in range(NB):
            t = g + i
            bf = (i + AHEAD) % NB

            @pl.when(t + AHEAD < hist)
            def _():
                @pl.when(t >= NB - AHEAD)
                def _():
                    drain_write(bf)
                fire_gather(t + AHEAD, bf)

            drain_gather(i)
            fire_write(t, i)

    for b in range(NB):
        drain_write(b)


def kernel(input_ids, weight):
    batch, hist = input_ids.shape
    info = plsc.get_sparse_core_info()
    nw = info.num_cores * info.num_subcores
    assert batch % (nw * BLOCK) == 0 or batch == nw * BLOCK
    assert hist % NB == 0

    ids_w = jnp.transpose(
        input_ids.astype(jnp.int32).reshape(nw, BLOCK, hist), (0, 2, 1))
    mesh = plsc.VectorSubcoreMesh(core_axis_name="c", subcore_axis_name="s")

    run = pl.kernel(
        functools.partial(_gather_body, info.num_cores, hist),
        out_type=jax.ShapeDtypeStruct((hist, batch, EMBED), jnp.float32),
        mesh=mesh,
        scratch_types=(
            [pltpu.VMEM((hist, BLOCK), jnp.int32)]
            + [pltpu.VMEM((BLOCK, EMBED), jnp.float32) for _ in range(NB)]
            + [pltpu.SemaphoreType.DMA for _ in range(2 * NB)]
        ),
    )
    out = run(ids_w, weight)
    return jnp.transpose(out, (1, 0, 2))
